# Initial kernel scaffold; baseline (speedup 1.0000x reference)
import jax
import jax.numpy as jnp
from jax.experimental import pallas as pl


def kernel(x, edge_index, edge_weight, Q_w, Q_b, W_w, W_b):
    raise NotImplementedError("write your pallas kernel here")


# speedup vs baseline: 4.6109x; 4.6109x over previous
import functools

import jax
import jax.numpy as jnp
from jax import lax
from jax.experimental import pallas as pl
from jax.experimental.pallas import tpu as pltpu
from jax.experimental.pallas import tpu_sc as plsc

N_NODES = 10000
N_EDGES = 320000
D = 128

NPAD = 10240
NC = 2
NS = 16
CHUNK = 80
E_PER_W = N_EDGES // (NC * NS)
NCHUNK = E_PER_W // CHUNK
ROWS_PER_TILE = NPAD // NS


def _tc_pre(x_ref, qwT_ref, qb_ref, out_ref):
    acc = jnp.dot(x_ref[...], qwT_ref[...], preferred_element_type=jnp.float32)
    out_ref[...] = jnp.maximum(acc + qb_ref[...], 0.0)


def _sc_body(nsrc_hbm, src_hbm, dst_hbm, w_hbm, acc_out, ws_out,
             srcv, dstv, wv, rows, zbuf, wszbuf, acc_sh, ws_sh):
    c = lax.axis_index("c")
    s = lax.axis_index("s")
    zero16 = jnp.zeros((16,), jnp.float32)

    def zrow(r, carry):
        for j in range(D // 16):
            zbuf[r, pl.ds(j * 16, 16)] = zero16
        return carry
    lax.fori_loop(0, CHUNK, zrow, 0)

    def zws(i, carry):
        wszbuf[pl.ds(i * 16, 16)] = zero16
        return carry
    lax.fori_loop(0, ROWS_PER_TILE // 16, zws, 0)

    base = s * ROWS_PER_TILE
    for j in range(ROWS_PER_TILE // CHUNK):
        pltpu.sync_copy(zbuf, acc_sh.at[pl.ds(base + j * CHUNK, CHUNK)])
    pltpu.sync_copy(wszbuf, ws_sh.at[pl.ds(base, ROWS_PER_TILE)])
    plsc.subcore_barrier()

    ebase = c * (N_EDGES // NC) + s * E_PER_W

    def chunk_body(i, carry):
        off = ebase + i * CHUNK
        pltpu.sync_copy(src_hbm.at[pl.ds(off, CHUNK)], srcv)
        pltpu.sync_copy(dst_hbm.at[pl.ds(off, CHUNK)], dstv)
        pltpu.sync_copy(w_hbm.at[pl.ds(off, CHUNK)], wv)
        pltpu.sync_copy(nsrc_hbm.at[srcv], rows)

        def grp_body(g, gcarry):
            wgrp = wv[pl.ds(g * 16, 16)]
            for t in range(16):
                wvec = jnp.full((16,), wgrp[t], jnp.float32)
                r = g * 16 + t
                for j in range(D // 16):
                    sl = pl.ds(j * 16, 16)
                    rows[r, sl] = rows[r, sl] * wvec
            return gcarry
        lax.fori_loop(0, CHUNK // 16, grp_body, 0)

        pltpu.sync_copy(rows, acc_sh.at[dstv], add=True)
        pltpu.sync_copy(wv, ws_sh.at[dstv], add=True)
        return carry
    lax.fori_loop(0, NCHUNK, chunk_body, 0)
    plsc.subcore_barrier()

    for j in range(ROWS_PER_TILE // CHUNK):
        sl = pl.ds(base + j * CHUNK, CHUNK)
        pltpu.sync_copy(acc_sh.at[sl], acc_out.at[c, sl])
    pltpu.sync_copy(ws_sh.at[pl.ds(base, ROWS_PER_TILE)],
                    ws_out.at[c, pl.ds(base, ROWS_PER_TILE)])


_sc_scatter = functools.partial(
    pl.kernel,
    out_type=(jax.ShapeDtypeStruct((NC, NPAD, D), jnp.float32),
              jax.ShapeDtypeStruct((NC, NPAD), jnp.float32)),
    mesh=plsc.VectorSubcoreMesh(core_axis_name="c", subcore_axis_name="s"),
    scratch_types=[
        pltpu.VMEM((CHUNK,), jnp.int32),
        pltpu.VMEM((CHUNK,), jnp.int32),
        pltpu.VMEM((CHUNK,), jnp.float32),
        pltpu.VMEM((CHUNK, D), jnp.float32),
        pltpu.VMEM((CHUNK, D), jnp.float32),
        pltpu.VMEM((ROWS_PER_TILE,), jnp.float32),
        pltpu.VMEM_SHARED((NPAD, D), jnp.float32),
        pltpu.VMEM_SHARED((NPAD,), jnp.float32),
    ],
)(_sc_body)


def _tc_post(acc_ref, ws_ref, x_ref, w1T_ref, w2T_ref, wb_ref, out_ref):
    n = acc_ref[0] + acc_ref[1]
    ws = jnp.maximum(ws_ref[0] + ws_ref[1], 1.0)
    h = n / ws
    z = jnp.dot(h, w1T_ref[...], preferred_element_type=jnp.float32)
    z = z + jnp.dot(x_ref[...], w2T_ref[...], preferred_element_type=jnp.float32)
    z = jnp.maximum(z + wb_ref[...], 0.0)
    norm = jnp.sqrt(jnp.sum(z * z, axis=1, keepdims=True))
    norm = jnp.where(norm == 0.0, 1.0, norm)
    out_ref[...] = z / norm


def kernel(x, edge_index, edge_weight, Q_w, Q_b, W_w, W_b):
    x_pad = jnp.pad(x, ((0, NPAD - N_NODES), (0, 0)))
    src = edge_index[0]
    dst = edge_index[1]
    w = edge_weight.astype(jnp.float32)

    nsrc = pl.pallas_call(
        _tc_pre,
        out_shape=jax.ShapeDtypeStruct((NPAD, D), jnp.float32),
    )(x_pad, Q_w.T, Q_b.reshape(1, D))

    acc, ws = _sc_scatter(nsrc, src, dst, w)

    out = pl.pallas_call(
        _tc_post,
        out_shape=jax.ShapeDtypeStruct((NPAD, D), jnp.float32),
    )(acc, ws.reshape(NC, NPAD, 1), x_pad,
      W_w[:, :D].T, W_w[:, D:].T, W_b.reshape(1, D))
    return out[:N_NODES]
